# baseline (speedup 1.0000x reference)
import functools

import jax
import jax.numpy as jnp
from jax import lax
from jax.experimental import pallas as pl
from jax.experimental.pallas import tpu as pltpu
from jax.experimental.pallas import tpu_sc as plsc

N = 10000
D = 256
H = 128
E = 160000
NC = 2
NT = 16
NP = 10240
PT = NP // NT
EP = 163840
EPT = EP // (NC * NT)
RPT = NP // NT

@functools.cache
def _mesh():
    return plsc.VectorSubcoreMesh(core_axis_name="c", subcore_axis_name="s",
                                  num_cores=NC, num_subcores=NT)


def _fill(ref, n, val, dtype):
    v = jnp.full((16,), val, dtype)

    def body(i, _):
        ref[pl.ds(i * 16, 16)] = v
        return 0

    lax.fori_loop(0, n // 16, body, 0)


def _rsqrt16(x):
    i = lax.bitcast_convert_type(x, jnp.int32)
    i = jnp.int32(0x5F3759DF) - lax.shift_right_logical(i, 1)
    y = lax.bitcast_convert_type(i, jnp.float32)
    for _ in range(3):
        y = y * (1.5 - 0.5 * x * y * y)
    return y


_DC = 2000
_NCH = 1024


@functools.cache
def _pre():
    return pl.kernel(
        _pre_body,
        out_type=(
            jax.ShapeDtypeStruct((EP,), jnp.float32),
            jax.ShapeDtypeStruct((NP,), jnp.float32),
        ),
        mesh=_mesh(),
        compiler_params=pltpu.CompilerParams(needs_layout_passes=False),
        scratch_types=[
            pltpu.VMEM((_DC,), jnp.int32),
            pltpu.VMEM((_DC,), jnp.float32),
            pltpu.VMEM((PT,), jnp.float32),
            pltpu.VMEM((NP,), jnp.float32),
            pltpu.VMEM((_NCH,), jnp.int32),
            pltpu.VMEM((_NCH,), jnp.int32),
            pltpu.VMEM((_NCH,), jnp.float32),
            pltpu.VMEM((_NCH,), jnp.float32),
            pltpu.VMEM_SHARED((NP,), jnp.float32),
        ],
    )


def _pre_body(row_hbm, col_hbm, w_hbm, norm_hbm, dis2_hbm,
         colv, wv, slicev, fullv, nrow, ncol, nw, nout, deg_sh):
    c = lax.axis_index("c")
    s = lax.axis_index("s")

    _fill(slicev, PT, 1.0, jnp.float32)
    pltpu.sync_copy(slicev, deg_sh.at[pl.ds(s * PT, PT)])
    plsc.subcore_barrier()

    def deg_chunk(k, _):
        base = s * (E // NT) + k * _DC
        pltpu.sync_copy(col_hbm.at[pl.ds(base, _DC)], colv)
        pltpu.sync_copy(w_hbm.at[pl.ds(base, _DC)], wv)
        pltpu.sync_copy(wv, deg_sh.at[colv], add=True)
        return 0

    lax.fori_loop(0, (E // NT) // _DC, deg_chunk, 0)
    plsc.subcore_barrier()

    pltpu.sync_copy(deg_sh, fullv)

    def dis_blk(i, _):
        d = fullv[pl.ds(i * 16, 16)]
        fullv[pl.ds(i * 16, 16)] = _rsqrt16(d)
        return 0

    lax.fori_loop(0, NP // 16, dis_blk, 0)

    @pl.when(c == 0)
    def _():
        def d2_blk(i, _):
            v = fullv[pl.ds(s * PT + i * 16, 16)]
            slicev[pl.ds(i * 16, 16)] = v * v
            return 0

        lax.fori_loop(0, PT // 16, d2_blk, 0)
        pltpu.sync_copy(slicev, dis2_hbm.at[pl.ds(s * PT, PT)])

    wid = s * NC + c

    def norm_chunk(k, _):
        base = wid * EPT + k * _NCH
        pltpu.sync_copy(row_hbm.at[pl.ds(base, _NCH)], nrow)
        pltpu.sync_copy(col_hbm.at[pl.ds(base, _NCH)], ncol)
        pltpu.sync_copy(w_hbm.at[pl.ds(base, _NCH)], nw)

        def grp(j, _):
            r16 = nrow[pl.ds(j * 16, 16)]
            c16 = ncol[pl.ds(j * 16, 16)]
            w16 = nw[pl.ds(j * 16, 16)]
            dr = plsc.load_gather(fullv, [r16])
            dc = plsc.load_gather(fullv, [c16])
            nout[pl.ds(j * 16, 16)] = dr * w16 * dc
            return 0

        lax.fori_loop(0, _NCH // 16, grp, 0)
        pltpu.sync_copy(nout, norm_hbm.at[pl.ds(base, _NCH)])
        return 0

    lax.fori_loop(0, EPT // _NCH, norm_chunk, 0)


_AC = 128
_SCE = 1024
_EPA = EP // NT
_NSUP = _EPA // _SCE
_CPS = _SCE // _AC
_ERB = EP // _AC
_ZR = 128


@functools.cache
def _agg():
    return pl.kernel(
        _agg_body,
        out_type=jax.ShapeDtypeStruct((NC * NP, H), jnp.float32),
        mesh=_mesh(),
        compiler_params=pltpu.CompilerParams(needs_layout_passes=False),
        scratch_types=[
            pltpu.VMEM((_CPS, _AC), jnp.int32),
            pltpu.VMEM((_CPS, _AC), jnp.int32),
            pltpu.VMEM((_CPS, _AC), jnp.int32),
            pltpu.VMEM((_CPS, _AC), jnp.int32),
            pltpu.VMEM((_CPS, _AC), jnp.float32),
            pltpu.VMEM((_CPS, _AC), jnp.float32),
            pltpu.VMEM((_AC, H), jnp.float32),
            pltpu.VMEM((_AC, H), jnp.float32),
            pltpu.VMEM_SHARED((NP, H), jnp.float32),
            pltpu.SemaphoreType.DMA,
            pltpu.SemaphoreType.DMA,
        ],
    )


def _agg_body(h_hbm, row_hbm, col_hbm, norm_hbm, out_hbm,
              rscA, rscB, cscA, cscB, nscA, nscB, buf0, buf1, acc,
              gsem0, gsem1):
    c = lax.axis_index("c")
    s = lax.axis_index("s")
    bufs = (buf0, buf1)
    gsems = (gsem0, gsem1)

    r0 = s * RPT

    def zfill(i, _):
        q = i // 8
        r = i % 8
        buf0[q, pl.ds(r * 16, 16)] = jnp.zeros((16,), jnp.float32)
        return 0

    lax.fori_loop(0, _ZR * (H // 16), zfill, 0)
    for t in range(RPT // _ZR):
        pltpu.sync_copy(buf0, acc.at[pl.ds(r0 + t * _ZR, _ZR)])
    plsc.subcore_barrier()

    erow0 = s * (_EPA // _AC)

    def load_sup(sc, rsc, csc, nsc):
        rb = erow0 + sc * _CPS
        pltpu.sync_copy(row_hbm.at[pl.ds(c * _ERB + rb, _CPS)], rsc)
        pltpu.sync_copy(col_hbm.at[pl.ds(rb, _CPS)], csc)
        pltpu.sync_copy(norm_hbm.at[pl.ds(rb, _CPS)], nsc)

    _NSS = 4
    _SSR = _AC // _NSS

    class _GD:
        def __init__(self, rsc, jj, p):
            self.rsc, self.jj, self.p = rsc, jj, p

        def start(self):
            for q in range(_NSS):
                pltpu.async_copy(
                    h_hbm.at[self.rsc.at[self.jj, pl.ds(q * _SSR, _SSR)]],
                    bufs[self.p].at[pl.ds(q * _SSR, _SSR)],
                    gsems[self.p])

        def wait(self):
            pltpu.make_async_copy(
                h_hbm.at[pl.ds(0, _AC)], bufs[self.p], gsems[self.p]).wait()

    def gd(rsc, jj, p):
        return _GD(rsc, jj, p)

    def do_chunk(rsc, csc, nsc, jj, p, issue_next):
        gd(rsc, jj, p).wait()
        issue_next()
        buf = bufs[p]

        @plsc.parallel_loop(0, _AC, unroll=4)
        def _(j):
            nv = plsc.load_gather(
                nsc,
                [jnp.full((16,), jj, jnp.int32), lax.broadcast(j, (16,))])
            for kk in range(H // 16):
                buf[j, pl.ds(kk * 16, 16)] = buf[j, pl.ds(kk * 16, 16)] * nv

        pltpu.sync_copy(buf, acc.at[csc.at[jj]], add=True)

    load_sup(0, rscA, cscA, nscA)
    gd(rscA, 0, 0).start()

    def outer(i, _):
        load_sup(2 * i + 1, rscB, cscB, nscB)
        for jj in range(_CPS):
            p = jj % 2
            if jj < _CPS - 1:
                nxt = lambda jj=jj, p=p: gd(rscA, jj + 1, 1 - p).start()
            else:
                nxt = lambda p=p: gd(rscB, 0, 1 - p).start()
            do_chunk(rscA, cscA, nscA, jj, p, nxt)
        load_sup(jnp.minimum(2 * i + 2, 2 * _NSUP // 2 - 1), rscA, cscA, nscA)
        for jj in range(_CPS):
            p = jj % 2
            if jj < _CPS - 1:
                nxt = lambda jj=jj, p=p: gd(rscB, jj + 1, 1 - p).start()
            else:
                def nxt(p=p):
                    @pl.when(i < _NSUP // 2 - 1)
                    def _():
                        gd(rscA, 0, 1 - p).start()
            do_chunk(rscB, cscB, nscB, jj, p, nxt)
        return 0

    lax.fori_loop(0, _NSUP // 2, outer, 0)
    plsc.subcore_barrier()

    ooff = c * NP
    for t in range(RPT // _ZR):
        pltpu.sync_copy(acc.at[pl.ds(r0 + t * _ZR, _ZR)], buf0)
        pltpu.sync_copy(buf0, out_hbm.at[pl.ds(ooff + r0 + t * _ZR, _ZR)])


_BM = 2000


def _mm_body(x_ref, w_ref, out_ref):
    h = jnp.dot(x_ref[...], w_ref[...], preferred_element_type=jnp.float32)
    out_ref[0] = h[:, :H]
    out_ref[1] = h[:, H:]


_mm0 = pl.pallas_call(
    _mm_body,
    grid=(N // _BM,),
    in_specs=[
        pl.BlockSpec((_BM, D), lambda i: (i, 0)),
        pl.BlockSpec((D, D), lambda i: (0, 0)),
    ],
    out_specs=pl.BlockSpec((2, _BM, H), lambda i: (0, i, 0)),
    out_shape=jax.ShapeDtypeStruct((2, N, H), jnp.float32),
)


def _make_stats(relu):
    def body(agg_ref, hs_ref, x0_ref, dis2_ref, b_ref, t_ref, sums_ref):
        i = pl.program_id(0)
        a = jnp.concatenate([agg_ref[0], agg_ref[1]], axis=1)
        h = jnp.concatenate([hs_ref[0], hs_ref[1]], axis=1)
        t = a + h * dis2_ref[...] + x0_ref[...] + b_ref[...]
        if relu:
            t = jnp.maximum(t, 0.0)
        t_ref[0] = t[:, :H]
        t_ref[1] = t[:, H:]
        ps = jnp.stack([jnp.sum(t, axis=0), jnp.sum(t * t, axis=0)])

        @pl.when(i == 0)
        def _():
            sums_ref[...] = ps

        @pl.when(i > 0)
        def _():
            sums_ref[...] = sums_ref[...] + ps

    return pl.pallas_call(
        body,
        grid=(N // _BM,),
        in_specs=[
            pl.BlockSpec((2, _BM, H), lambda i: (0, i, 0)),
            pl.BlockSpec((2, _BM, H), lambda i: (0, i, 0)),
            pl.BlockSpec((_BM, D), lambda i: (i, 0)),
            pl.BlockSpec((_BM, 1), lambda i: (i, 0)),
            pl.BlockSpec((1, D), lambda i: (0, 0)),
        ],
        out_specs=[
            pl.BlockSpec((2, _BM, H), lambda i: (0, i, 0)),
            pl.BlockSpec((2, D), lambda i: (0, 0)),
        ],
        out_shape=[
            jax.ShapeDtypeStruct((2, N, H), jnp.float32),
            jax.ShapeDtypeStruct((2, D), jnp.float32),
        ],
    )


def _bn_from_sums(t, sums_ref, g_ref, be_ref):
    m = sums_ref[0:1, :] * (1.0 / N)
    v = sums_ref[1:2, :] * (1.0 / N) - m * m
    inv = lax.rsqrt(v + 1e-5)
    return (t - m) * inv * g_ref[...] + be_ref[...]


def _apply_mm_body(t_ref, sums_ref, g_ref, be_ref, w_ref, out_ref):
    t = jnp.concatenate([t_ref[0], t_ref[1]], axis=1)
    xn = _bn_from_sums(t, sums_ref, g_ref, be_ref)
    h = jnp.dot(xn, w_ref[...], preferred_element_type=jnp.float32)
    out_ref[0] = h[:, :H]
    out_ref[1] = h[:, H:]


_apply_mm = pl.pallas_call(
    _apply_mm_body,
    grid=(N // _BM,),
    in_specs=[
        pl.BlockSpec((2, _BM, H), lambda i: (0, i, 0)),
        pl.BlockSpec((2, D), lambda i: (0, 0)),
        pl.BlockSpec((1, D), lambda i: (0, 0)),
        pl.BlockSpec((1, D), lambda i: (0, 0)),
        pl.BlockSpec((D, D), lambda i: (0, 0)),
    ],
    out_specs=pl.BlockSpec((2, _BM, H), lambda i: (0, i, 0)),
    out_shape=jax.ShapeDtypeStruct((2, N, H), jnp.float32),
)


def _apply_fin_body(t_ref, sums_ref, g_ref, be_ref, out_ref):
    t = jnp.concatenate([t_ref[0], t_ref[1]], axis=1)
    out_ref[...] = _bn_from_sums(t, sums_ref, g_ref, be_ref)


_apply_fin = pl.pallas_call(
    _apply_fin_body,
    grid=(N // _BM,),
    in_specs=[
        pl.BlockSpec((2, _BM, H), lambda i: (0, i, 0)),
        pl.BlockSpec((2, D), lambda i: (0, 0)),
        pl.BlockSpec((1, D), lambda i: (0, 0)),
        pl.BlockSpec((1, D), lambda i: (0, 0)),
    ],
    out_specs=pl.BlockSpec((_BM, D), lambda i: (i, 0)),
    out_shape=jax.ShapeDtypeStruct((N, D), jnp.float32),
)

_stats_relu = _make_stats(True)
_stats_lin = _make_stats(False)


def kernel(x, edge_index, edge_weight, W0, b0, g0, be0,
           W1, b1, g1, be1, W2, b2, g2, be2):
    row = edge_index[0]
    col = edge_index[1]
    pad = EP - E
    rowp = jnp.concatenate([row, jnp.zeros((pad,), jnp.int32)])
    colp = jnp.concatenate([col, jnp.zeros((pad,), jnp.int32)])
    wp = jnp.concatenate([edge_weight, jnp.zeros((pad,), jnp.float32)])

    row2 = jnp.concatenate([rowp, rowp + N]).reshape(2 * _ERB, _AC)
    col2 = colp.reshape(_ERB, _AC)

    norm, dis2p = _pre()(rowp, colp, wp)
    dis2 = dis2p[:N].reshape(N, 1)

    hs = _mm0(x, W0)
    Ws = [W1, W2, None]
    bs = [b0, b1, b2]
    gs = [g0, g1, g2]
    bes = [be0, be1, be2]
    out = None
    for i in range(3):
        agg = _agg()(hs.reshape(NC * N, H), row2, col2,
                     norm.reshape(_ERB, _AC))
        stats = _stats_relu if i != 1 else _stats_lin
        t, sums = stats(agg.reshape(2, NP, H), hs, x, dis2, bs[i].reshape(1, D))
        if i < 2:
            hs = _apply_mm(t, sums, gs[i].reshape(1, D),
                           bes[i].reshape(1, D), Ws[i])
        else:
            out = _apply_fin(t, sums, gs[i].reshape(1, D), bes[i].reshape(1, D))
    return out


# speedup vs baseline: 1.0444x; 1.0444x over previous
import functools

import jax
import jax.numpy as jnp
from jax import lax
from jax.experimental import pallas as pl
from jax.experimental.pallas import tpu as pltpu
from jax.experimental.pallas import tpu_sc as plsc

N = 10000
D = 256
H = 128
E = 160000
NC = 2
NT = 16
NP = 10240
PT = NP // NT
EP = 163840
EPT = EP // (NC * NT)
RPT = NP // NT

@functools.cache
def _mesh():
    return plsc.VectorSubcoreMesh(core_axis_name="c", subcore_axis_name="s",
                                  num_cores=NC, num_subcores=NT)


def _fill(ref, n, val, dtype):
    v = jnp.full((16,), val, dtype)

    def body(i, _):
        ref[pl.ds(i * 16, 16)] = v
        return 0

    lax.fori_loop(0, n // 16, body, 0)


def _rsqrt16(x):
    i = lax.bitcast_convert_type(x, jnp.int32)
    i = jnp.int32(0x5F3759DF) - lax.shift_right_logical(i, 1)
    y = lax.bitcast_convert_type(i, jnp.float32)
    for _ in range(3):
        y = y * (1.5 - 0.5 * x * y * y)
    return y


_DC = 2000
_NCH = 1024


@functools.cache
def _pre():
    return pl.kernel(
        _pre_body,
        out_type=(
            jax.ShapeDtypeStruct((EP,), jnp.float32),
            jax.ShapeDtypeStruct((NP,), jnp.float32),
        ),
        mesh=_mesh(),
        compiler_params=pltpu.CompilerParams(needs_layout_passes=False),
        scratch_types=[
            pltpu.VMEM((_DC,), jnp.int32),
            pltpu.VMEM((_DC,), jnp.float32),
            pltpu.VMEM((PT,), jnp.float32),
            pltpu.VMEM((NP,), jnp.float32),
            pltpu.VMEM((_NCH,), jnp.int32),
            pltpu.VMEM((_NCH,), jnp.int32),
            pltpu.VMEM((_NCH,), jnp.float32),
            pltpu.VMEM((_NCH,), jnp.float32),
            pltpu.VMEM_SHARED((NP,), jnp.float32),
        ],
    )


def _pre_body(row_hbm, col_hbm, w_hbm, norm_hbm, dis2_hbm,
         colv, wv, slicev, fullv, nrow, ncol, nw, nout, deg_sh):
    c = lax.axis_index("c")
    s = lax.axis_index("s")

    _fill(slicev, PT, 1.0, jnp.float32)
    pltpu.sync_copy(slicev, deg_sh.at[pl.ds(s * PT, PT)])
    plsc.subcore_barrier()

    def deg_chunk(k, _):
        base = s * (E // NT) + k * _DC
        pltpu.sync_copy(col_hbm.at[pl.ds(base, _DC)], colv)
        pltpu.sync_copy(w_hbm.at[pl.ds(base, _DC)], wv)
        pltpu.sync_copy(wv, deg_sh.at[colv], add=True)
        return 0

    lax.fori_loop(0, (E // NT) // _DC, deg_chunk, 0)
    plsc.subcore_barrier()

    pltpu.sync_copy(deg_sh, fullv)

    def dis_blk(i, _):
        d = fullv[pl.ds(i * 16, 16)]
        fullv[pl.ds(i * 16, 16)] = _rsqrt16(d)
        return 0

    lax.fori_loop(0, NP // 16, dis_blk, 0)

    @pl.when(c == 0)
    def _():
        def d2_blk(i, _):
            v = fullv[pl.ds(s * PT + i * 16, 16)]
            slicev[pl.ds(i * 16, 16)] = v * v
            return 0

        lax.fori_loop(0, PT // 16, d2_blk, 0)
        pltpu.sync_copy(slicev, dis2_hbm.at[pl.ds(s * PT, PT)])

    wid = s * NC + c

    def norm_chunk(k, _):
        base = wid * EPT + k * _NCH
        pltpu.sync_copy(row_hbm.at[pl.ds(base, _NCH)], nrow)
        pltpu.sync_copy(col_hbm.at[pl.ds(base, _NCH)], ncol)
        pltpu.sync_copy(w_hbm.at[pl.ds(base, _NCH)], nw)

        def grp(j, _):
            r16 = nrow[pl.ds(j * 16, 16)]
            c16 = ncol[pl.ds(j * 16, 16)]
            w16 = nw[pl.ds(j * 16, 16)]
            dr = plsc.load_gather(fullv, [r16])
            dc = plsc.load_gather(fullv, [c16])
            nout[pl.ds(j * 16, 16)] = dr * w16 * dc
            return 0

        lax.fori_loop(0, _NCH // 16, grp, 0)
        pltpu.sync_copy(nout, norm_hbm.at[pl.ds(base, _NCH)])
        return 0

    lax.fori_loop(0, EPT // _NCH, norm_chunk, 0)


_AC = 200
_NCHUNK = (E // NT) // _AC
_ZR = 128


@functools.cache
def _agg():
    return pl.kernel(
        _agg_body,
        out_type=jax.ShapeDtypeStruct((NC * NP, H), jnp.float32),
        mesh=_mesh(),
        compiler_params=pltpu.CompilerParams(needs_layout_passes=False),
        scratch_types=[
            pltpu.VMEM((_AC,), jnp.int32),
            pltpu.VMEM((_AC,), jnp.int32),
            pltpu.VMEM((_AC,), jnp.float32),
            pltpu.VMEM((_AC, H), jnp.float32),
            pltpu.VMEM((_ZR, H), jnp.float32),
            pltpu.VMEM_SHARED((NP, H), jnp.float32),
            pltpu.SemaphoreType.DMA,
        ],
    )


def _agg_body(h_hbm, row_hbm, col_hbm, norm_hbm, out_hbm,
              rowv, colv, nrmv, buf, stage, acc, sem):
    c = lax.axis_index("c")
    s = lax.axis_index("s")

    def zfill(i, _):
        q = i // 8
        r = i % 8
        stage[q, pl.ds(r * 16, 16)] = jnp.zeros((16,), jnp.float32)
        return 0

    lax.fori_loop(0, _ZR * (H // 16), zfill, 0)
    r0 = s * RPT
    for t in range(RPT // _ZR):
        pltpu.sync_copy(stage, acc.at[pl.ds(r0 + t * _ZR, _ZR)])
    plsc.subcore_barrier()

    ooff = c * NP

    def chunk(k, _):
        base = s * (E // NT) + k * _AC
        pltpu.sync_copy(row_hbm.at[pl.ds(c * E + base, _AC)], rowv)
        pltpu.sync_copy(col_hbm.at[pl.ds(base, _AC)], colv)
        pltpu.sync_copy(norm_hbm.at[pl.ds(base, _AC)], nrmv)

        pltpu.async_copy(h_hbm.at[rowv], buf, sem).wait()

        def scale(j, _):
            idx = lax.broadcast(j, (16,))
            nv = plsc.load_gather(nrmv, [idx])
            for kk in range(H // 16):
                buf[j, pl.ds(kk * 16, 16)] = buf[j, pl.ds(kk * 16, 16)] * nv
            return 0

        lax.fori_loop(0, _AC, scale, 0)

        pltpu.sync_copy(buf, acc.at[colv], add=True)
        return 0

    lax.fori_loop(0, _NCHUNK, chunk, 0)
    plsc.subcore_barrier()

    for t in range(RPT // _ZR):
        pltpu.sync_copy(acc.at[pl.ds(r0 + t * _ZR, _ZR)], stage)
        pltpu.sync_copy(stage, out_hbm.at[pl.ds(ooff + r0 + t * _ZR, _ZR)])


_BM = 2000


def _mm_body(x_ref, w_ref, out_ref):
    h = jnp.dot(x_ref[...], w_ref[...], preferred_element_type=jnp.float32)
    out_ref[0] = h[:, :H]
    out_ref[1] = h[:, H:]


_mm0 = pl.pallas_call(
    _mm_body,
    grid=(N // _BM,),
    in_specs=[
        pl.BlockSpec((_BM, D), lambda i: (i, 0)),
        pl.BlockSpec((D, D), lambda i: (0, 0)),
    ],
    out_specs=pl.BlockSpec((2, _BM, H), lambda i: (0, i, 0)),
    out_shape=jax.ShapeDtypeStruct((2, N, H), jnp.float32),
)


def _make_stats(relu):
    def body(agg_ref, hs_ref, x0_ref, dis2_ref, b_ref, t_ref, sums_ref):
        i = pl.program_id(0)
        a = jnp.concatenate([agg_ref[0], agg_ref[1]], axis=1)
        h = jnp.concatenate([hs_ref[0], hs_ref[1]], axis=1)
        t = a + h * dis2_ref[...] + x0_ref[...] + b_ref[...]
        if relu:
            t = jnp.maximum(t, 0.0)
        t_ref[0] = t[:, :H]
        t_ref[1] = t[:, H:]
        ps = jnp.stack([jnp.sum(t, axis=0), jnp.sum(t * t, axis=0)])

        @pl.when(i == 0)
        def _():
            sums_ref[...] = ps

        @pl.when(i > 0)
        def _():
            sums_ref[...] = sums_ref[...] + ps

    return pl.pallas_call(
        body,
        grid=(N // _BM,),
        in_specs=[
            pl.BlockSpec((2, _BM, H), lambda i: (0, i, 0)),
            pl.BlockSpec((2, _BM, H), lambda i: (0, i, 0)),
            pl.BlockSpec((_BM, D), lambda i: (i, 0)),
            pl.BlockSpec((_BM, 1), lambda i: (i, 0)),
            pl.BlockSpec((1, D), lambda i: (0, 0)),
        ],
        out_specs=[
            pl.BlockSpec((2, _BM, H), lambda i: (0, i, 0)),
            pl.BlockSpec((2, D), lambda i: (0, 0)),
        ],
        out_shape=[
            jax.ShapeDtypeStruct((2, N, H), jnp.float32),
            jax.ShapeDtypeStruct((2, D), jnp.float32),
        ],
    )


def _bn_from_sums(t, sums_ref, g_ref, be_ref):
    m = sums_ref[0:1, :] * (1.0 / N)
    v = sums_ref[1:2, :] * (1.0 / N) - m * m
    inv = lax.rsqrt(v + 1e-5)
    return (t - m) * inv * g_ref[...] + be_ref[...]


def _apply_mm_body(t_ref, sums_ref, g_ref, be_ref, w_ref, out_ref):
    t = jnp.concatenate([t_ref[0], t_ref[1]], axis=1)
    xn = _bn_from_sums(t, sums_ref, g_ref, be_ref)
    h = jnp.dot(xn, w_ref[...], preferred_element_type=jnp.float32)
    out_ref[0] = h[:, :H]
    out_ref[1] = h[:, H:]


_apply_mm = pl.pallas_call(
    _apply_mm_body,
    grid=(N // _BM,),
    in_specs=[
        pl.BlockSpec((2, _BM, H), lambda i: (0, i, 0)),
        pl.BlockSpec((2, D), lambda i: (0, 0)),
        pl.BlockSpec((1, D), lambda i: (0, 0)),
        pl.BlockSpec((1, D), lambda i: (0, 0)),
        pl.BlockSpec((D, D), lambda i: (0, 0)),
    ],
    out_specs=pl.BlockSpec((2, _BM, H), lambda i: (0, i, 0)),
    out_shape=jax.ShapeDtypeStruct((2, N, H), jnp.float32),
)


def _apply_fin_body(t_ref, sums_ref, g_ref, be_ref, out_ref):
    t = jnp.concatenate([t_ref[0], t_ref[1]], axis=1)
    out_ref[...] = _bn_from_sums(t, sums_ref, g_ref, be_ref)


_apply_fin = pl.pallas_call(
    _apply_fin_body,
    grid=(N // _BM,),
    in_specs=[
        pl.BlockSpec((2, _BM, H), lambda i: (0, i, 0)),
        pl.BlockSpec((2, D), lambda i: (0, 0)),
        pl.BlockSpec((1, D), lambda i: (0, 0)),
        pl.BlockSpec((1, D), lambda i: (0, 0)),
    ],
    out_specs=pl.BlockSpec((_BM, D), lambda i: (i, 0)),
    out_shape=jax.ShapeDtypeStruct((N, D), jnp.float32),
)

_stats_relu = _make_stats(True)
_stats_lin = _make_stats(False)


def kernel(x, edge_index, edge_weight, W0, b0, g0, be0,
           W1, b1, g1, be1, W2, b2, g2, be2):
    row = edge_index[0]
    col = edge_index[1]
    pad = EP - E
    rowp = jnp.concatenate([row, jnp.zeros((pad,), jnp.int32)])
    colp = jnp.concatenate([col, jnp.zeros((pad,), jnp.int32)])
    wp = jnp.concatenate([edge_weight, jnp.zeros((pad,), jnp.float32)])

    row2 = jnp.concatenate([row, row + N])

    norm, dis2p = _pre()(rowp, colp, wp)
    dis2 = dis2p[:N].reshape(N, 1)

    hs = _mm0(x, W0)
    Ws = [W1, W2, None]
    bs = [b0, b1, b2]
    gs = [g0, g1, g2]
    bes = [be0, be1, be2]
    out = None
    for i in range(3):
        agg = _agg()(hs.reshape(NC * N, H), row2, col, norm)
        stats = _stats_relu if i != 1 else _stats_lin
        t, sums = stats(agg.reshape(2, NP, H), hs, x, dis2, bs[i].reshape(1, D))
        if i < 2:
            hs = _apply_mm(t, sums, gs[i].reshape(1, D),
                           bes[i].reshape(1, D), Ws[i])
        else:
            out = _apply_fin(t, sums, gs[i].reshape(1, D), bes[i].reshape(1, D))
    return out
